# baseline (speedup 1.0000x reference)
import functools

import jax
import jax.numpy as jnp
from jax import lax
from jax.experimental import pallas as pl
from jax.experimental.pallas import tpu as pltpu
from jax.experimental.pallas import tpu_sc as plsc

_LANES = 16
_TILE = 128
_NSLOT = 8


def _make_kernel(batch, dim, num_cores, num_subcores):
    num_workers = num_cores * num_subcores
    rows_per_worker = batch // num_workers
    n_groups = rows_per_worker // _LANES
    assert batch % (num_workers * _LANES) == 0
    assert dim == 2 * _LANES

    mesh = plsc.VectorSubcoreMesh(
        core_axis_name="c", subcore_axis_name="s",
        num_cores=num_cores, num_subcores=num_subcores,
    )

    @functools.partial(
        pl.kernel,
        out_type=jax.ShapeDtypeStruct((batch,), jnp.float32),
        mesh=mesh,
        compiler_params=pltpu.CompilerParams(needs_layout_passes=False),
        scratch_types=[
            pltpu.VMEM((rows_per_worker,), jnp.int32),
            pltpu.VMEM((rows_per_worker,), jnp.int32),
            pltpu.VMEM((_NSLOT, dim, _TILE), jnp.float32),
            pltpu.VMEM((_NSLOT, dim, _TILE), jnp.float32),
            pltpu.VMEM((_LANES, _LANES), jnp.float32),
            pltpu.VMEM((40, _LANES), jnp.float32),
            pltpu.VMEM((rows_per_worker,), jnp.float32),
        ] + [pltpu.SemaphoreType.DMA] * _NSLOT,
    )
    def net_kernel(users_hbm, items_hbm, utabt_hbm, itabt_hbm, wb_hbm, out_hbm,
                   uidx, iidx, utiles, itiles, stage, wbv, outv, *sems):
        wid = lax.axis_index("s") * num_cores + lax.axis_index("c")
        base = pl.multiple_of(wid * rows_per_worker, rows_per_worker)

        pltpu.sync_copy(users_hbm.at[pl.ds(base, rows_per_worker)], uidx)
        pltpu.sync_copy(items_hbm.at[pl.ds(base, rows_per_worker)], iidx)
        pltpu.sync_copy(wb_hbm, wbv)

        iota16 = lax.iota(jnp.int32, _LANES)
        zeros16 = jnp.zeros((_LANES,), jnp.int32)

        def lane_scalar(vec, k):
            return jnp.sum(jnp.where(iota16 == k, vec, 0))

        w_lo = plsc.load_gather(wbv, [iota16, zeros16])
        w_hi = plsc.load_gather(wbv, [iota16 + _LANES, zeros16])

        def fire(slot, bu, bi):
            offu = pl.multiple_of(bu * _TILE, _TILE)
            offi = pl.multiple_of(bi * _TILE, _TILE)
            for co in range(dim // 8):
                rs = pl.ds(co * 8, 8)
                pltpu.async_copy(
                    utabt_hbm.at[rs, pl.ds(offu, _TILE)],
                    utiles.at[slot].at[rs], sems[slot])
                pltpu.async_copy(
                    itabt_hbm.at[rs, pl.ds(offi, _TILE)],
                    itiles.at[slot].at[rs], sems[slot])

        def wait_slot(slot):
            dummy = utabt_hbm.at[:, pl.ds(0, _TILE)]
            pltpu.make_async_copy(dummy, utiles.at[slot], sems[slot]).wait()
            pltpu.make_async_copy(dummy, itiles.at[slot], sems[slot]).wait()

        rvu0 = uidx[pl.ds(0, _LANES)]
        rvi0 = iidx[pl.ds(0, _LANES)]
        for s in range(_NSLOT):
            fire(s,
                 lane_scalar(lax.shift_right_logical(rvu0, 7), s),
                 lane_scalar(lax.shift_right_logical(rvi0, 7), s))

        def group_body(g, carry):
            goff = pl.multiple_of(g * _LANES, _LANES)
            rvu = uidx[pl.ds(goff, _LANES)]
            rvi = iidx[pl.ds(goff, _LANES)]
            col_u = lax.bitwise_and(rvu, 127)
            col_i = lax.bitwise_and(rvi, 127)
            blk_u = lax.shift_right_logical(rvu, 7)
            blk_i = lax.shift_right_logical(rvi, 7)
            gnoff = pl.multiple_of(
                lax.rem(g + 1, n_groups) * _LANES, _LANES)
            blk_un = lax.shift_right_logical(uidx[pl.ds(gnoff, _LANES)], 7)
            blk_in = lax.shift_right_logical(iidx[pl.ds(gnoff, _LANES)], 7)

            for k in range(_LANES):
                slot = k % _NSLOT
                cu_s = lane_scalar(col_u, k)
                ci_s = lane_scalar(col_i, k)
                if k < _NSLOT:
                    bu_s = lane_scalar(blk_u, k + _NSLOT)
                    bi_s = lane_scalar(blk_i, k + _NSLOT)
                else:
                    bu_s = lane_scalar(blk_un, k - _NSLOT)
                    bi_s = lane_scalar(blk_in, k - _NSLOT)
                wait_slot(slot)
                cu = zeros16 + cu_s
                ci = zeros16 + ci_s
                u1 = plsc.load_gather(utiles.at[slot], [iota16, cu])
                u2 = plsc.load_gather(utiles.at[slot], [iota16 + _LANES, cu])
                i1 = plsc.load_gather(itiles.at[slot], [iota16, ci])
                i2 = plsc.load_gather(itiles.at[slot], [iota16 + _LANES, ci])
                stage[k, :] = u1 * i1 * w_lo + u2 * i2 * w_hi
                fire(slot, bu_s, bi_s)

            acc = wbv[dim]
            for c in range(_LANES):
                diag = plsc.load_gather(
                    stage, [iota16, lax.bitwise_and(iota16 + c, _LANES - 1)])
                acc = acc + diag
            y = 1.0 / (1.0 + jnp.exp(-acc))
            outv[pl.ds(goff, _LANES)] = y
            return carry

        lax.fori_loop(0, n_groups, group_body, 0)

        for s in range(_NSLOT):
            wait_slot(s)

        pltpu.sync_copy(outv, out_hbm.at[pl.ds(base, rows_per_worker)])

    return net_kernel


def kernel(users_input, items_input, user_table, item_table, ll_weight, ll_bias):
    batch = users_input.shape[0]
    dim = user_table.shape[1]
    utabt = user_table.T
    itabt = item_table.T
    wb = jnp.concatenate([
        ll_weight.reshape(-1).astype(jnp.float32),
        ll_bias.astype(jnp.float32),
        jnp.zeros((7,), jnp.float32),
    ])
    wb = jnp.broadcast_to(wb[:, None], (40, _LANES))
    net = _make_kernel(batch, dim, num_cores=2, num_subcores=16)
    return net(users_input, items_input, utabt, itabt, wb)


# speedup vs baseline: 1.0085x; 1.0085x over previous
import functools

import jax
import jax.numpy as jnp
from jax import lax
from jax.experimental import pallas as pl
from jax.experimental.pallas import tpu as pltpu
from jax.experimental.pallas import tpu_sc as plsc

_LANES = 16
_TILE = 128
_NSLOT = 8


def _make_kernel(batch, dim, num_cores, num_subcores):
    num_workers = num_cores * num_subcores
    rows_per_worker = batch // num_workers
    n_groups = rows_per_worker // _LANES
    assert batch % (num_workers * _LANES) == 0
    assert dim == 2 * _LANES

    mesh = plsc.VectorSubcoreMesh(
        core_axis_name="c", subcore_axis_name="s",
        num_cores=num_cores, num_subcores=num_subcores,
    )

    @functools.partial(
        pl.kernel,
        out_type=jax.ShapeDtypeStruct((batch,), jnp.float32),
        mesh=mesh,
        compiler_params=pltpu.CompilerParams(needs_layout_passes=False),
        scratch_types=[
            pltpu.VMEM((rows_per_worker,), jnp.int32),
            pltpu.VMEM((rows_per_worker,), jnp.int32),
            pltpu.VMEM((_NSLOT, dim, _TILE), jnp.float32),
            pltpu.VMEM((_NSLOT, dim, _TILE), jnp.float32),
            pltpu.VMEM((_LANES, _LANES), jnp.float32),
            pltpu.VMEM((40, _LANES), jnp.float32),
            pltpu.VMEM((rows_per_worker,), jnp.float32),
        ] + [pltpu.SemaphoreType.DMA] * _NSLOT,
    )
    def net_kernel(users_hbm, items_hbm, utabt_hbm, itabt_hbm, wb_hbm, out_hbm,
                   uidx, iidx, utiles, itiles, stage, wbv, outv, *sems):
        wid = lax.axis_index("s") * num_cores + lax.axis_index("c")
        base = pl.multiple_of(wid * rows_per_worker, rows_per_worker)

        pltpu.sync_copy(users_hbm.at[pl.ds(base, rows_per_worker)], uidx)
        pltpu.sync_copy(items_hbm.at[pl.ds(base, rows_per_worker)], iidx)
        pltpu.sync_copy(wb_hbm, wbv)

        iota16 = lax.iota(jnp.int32, _LANES)
        zeros16 = jnp.zeros((_LANES,), jnp.int32)

        def lane_scalar(vec, k):
            return jnp.sum(jnp.where(iota16 == k, vec, 0))

        w_lo = plsc.load_gather(wbv, [iota16, zeros16])
        w_hi = plsc.load_gather(wbv, [iota16 + _LANES, zeros16])

        def fire(slot, bu, bi):
            offu = pl.multiple_of(bu * _TILE, _TILE)
            offi = pl.multiple_of(bi * _TILE, _TILE)
            pltpu.async_copy(
                utabt_hbm.at[:, pl.ds(offu, _TILE)], utiles.at[slot], sems[slot])
            pltpu.async_copy(
                itabt_hbm.at[:, pl.ds(offi, _TILE)], itiles.at[slot], sems[slot])

        def wait_slot(slot):
            dummy = utabt_hbm.at[:, pl.ds(0, _TILE)]
            pltpu.make_async_copy(dummy, utiles.at[slot], sems[slot]).wait()
            pltpu.make_async_copy(dummy, itiles.at[slot], sems[slot]).wait()

        rvu0 = uidx[pl.ds(0, _LANES)]
        rvi0 = iidx[pl.ds(0, _LANES)]
        for s in range(_NSLOT):
            fire(s,
                 lane_scalar(lax.shift_right_logical(rvu0, 7), s),
                 lane_scalar(lax.shift_right_logical(rvi0, 7), s))

        def group_body(g, carry):
            goff = pl.multiple_of(g * _LANES, _LANES)
            rvu = uidx[pl.ds(goff, _LANES)]
            rvi = iidx[pl.ds(goff, _LANES)]
            col_u = lax.bitwise_and(rvu, 127)
            col_i = lax.bitwise_and(rvi, 127)
            blk_u = lax.shift_right_logical(rvu, 7)
            blk_i = lax.shift_right_logical(rvi, 7)
            gnoff = pl.multiple_of(
                lax.rem(g + 1, n_groups) * _LANES, _LANES)
            blk_un = lax.shift_right_logical(uidx[pl.ds(gnoff, _LANES)], 7)
            blk_in = lax.shift_right_logical(iidx[pl.ds(gnoff, _LANES)], 7)

            for k in range(_LANES):
                slot = k % _NSLOT
                cu_s = lane_scalar(col_u, k)
                ci_s = lane_scalar(col_i, k)
                if k < _NSLOT:
                    bu_s = lane_scalar(blk_u, k + _NSLOT)
                    bi_s = lane_scalar(blk_i, k + _NSLOT)
                else:
                    bu_s = lane_scalar(blk_un, k - _NSLOT)
                    bi_s = lane_scalar(blk_in, k - _NSLOT)
                wait_slot(slot)
                cu = zeros16 + cu_s
                ci = zeros16 + ci_s
                u1 = plsc.load_gather(utiles.at[slot], [iota16, cu])
                u2 = plsc.load_gather(utiles.at[slot], [iota16 + _LANES, cu])
                i1 = plsc.load_gather(itiles.at[slot], [iota16, ci])
                i2 = plsc.load_gather(itiles.at[slot], [iota16 + _LANES, ci])
                stage[k, :] = u1 * i1 * w_lo + u2 * i2 * w_hi
                fire(slot, bu_s, bi_s)

            acc = wbv[dim]
            for c in range(_LANES):
                diag = plsc.load_gather(
                    stage, [iota16, lax.bitwise_and(iota16 + c, _LANES - 1)])
                acc = acc + diag
            y = 1.0 / (1.0 + jnp.exp(-acc))
            outv[pl.ds(goff, _LANES)] = y
            return carry

        lax.fori_loop(0, n_groups, group_body, 0)

        for s in range(_NSLOT):
            wait_slot(s)

        pltpu.sync_copy(outv, out_hbm.at[pl.ds(base, rows_per_worker)])

    return net_kernel


def kernel(users_input, items_input, user_table, item_table, ll_weight, ll_bias):
    batch = users_input.shape[0]
    dim = user_table.shape[1]
    utabt = user_table.T
    itabt = item_table.T
    wb = jnp.concatenate([
        ll_weight.reshape(-1).astype(jnp.float32),
        ll_bias.astype(jnp.float32),
        jnp.zeros((7,), jnp.float32),
    ])
    wb = jnp.broadcast_to(wb[:, None], (40, _LANES))
    net = _make_kernel(batch, dim, num_cores=2, num_subcores=16)
    return net(users_input, items_input, utabt, itabt, wb)
